# baseline (speedup 1.0000x reference)
import functools

import jax
import jax.numpy as jnp
from jax import lax
from jax.experimental import pallas as pl
from jax.experimental.pallas import tpu as pltpu
from jax.experimental.pallas import tpu_sc as plsc

_N = 10000
_E = 320000
_D = 128
_CHUNK = 128
_NCHUNKS = _E // _CHUNK
_NP = 10240
_NC = 2
_NS = 16

_f32 = jnp.float32


_EPAD = 2560 * _CHUNK
_CPT = (_EPAD // _CHUNK) // (_NC * _NS)


def _sc_partials(feat_src, src2d, dst2d):
  mesh = plsc.VectorSubcoreMesh(core_axis_name="c", subcore_axis_name="s")
  rows_per_tile = _NP // _NS

  @functools.partial(
      pl.kernel,
      mesh=mesh,
      out_type=(
          jax.ShapeDtypeStruct((_NC, _NP, _D), _f32),
          jax.ShapeDtypeStruct((_NC, _NP), _f32),
      ),
      scratch_types=[
          pltpu.VMEM((_CHUNK, _D), _f32),
          pltpu.VMEM((_CHUNK, _D), _f32),
          pltpu.VMEM((_CPT // 2, _CHUNK), jnp.int32),
          pltpu.VMEM((_CPT // 2, _CHUNK), jnp.int32),
          pltpu.VMEM((_CHUNK,), _f32),
          pltpu.VMEM((rows_per_tile,), _f32),
          pltpu.VMEM_SHARED((_NP, _D), _f32),
          pltpu.VMEM_SHARED((_NP,), _f32),
          pltpu.SemaphoreType.DMA,
          pltpu.SemaphoreType.DMA,
          pltpu.SemaphoreType.DMA,
          pltpu.SemaphoreType.DMA,
          pltpu.SemaphoreType.DMA,
          pltpu.SemaphoreType.DMA,
      ],
  )
  def body(feat_hbm, src_hbm, dst_hbm, zeros_hbm, psum_out, pdeg_out,
           rbuf0, rbuf1, sidx_all, didx_all, ones, zdeg, acc, deg,
           gsem0, gsem1, ssem0, ssem1, dsem0, dsem1):
    c = lax.axis_index("c")
    s = lax.axis_index("s")
    t = c * _NS + s

    for i in range(_CHUNK // 16):
      ones[pl.ds(16 * i, 16)] = jnp.full((16,), 1.0, _f32)
    for i in range(rows_per_tile // 16):
      zdeg[pl.ds(16 * i, 16)] = jnp.zeros((16,), _f32)

    base = s * rows_per_tile
    pltpu.sync_copy(zeros_hbm.at[pl.ds(base, rows_per_tile), :],
                    acc.at[pl.ds(base, rows_per_tile), :])
    pltpu.sync_copy(zdeg, deg.at[pl.ds(s * rows_per_tile, rows_per_tile)])
    plsc.subcore_barrier()

    rbufs = (rbuf0, rbuf1)
    gsems = (gsem0, gsem1)
    ssems = (ssem0, ssem1)
    dsems = (dsem0, dsem1)
    cpp = _CPT // 2

    for ph in range(2):
      pltpu.sync_copy(src_hbm.at[pl.ds(t * _CPT + ph * cpp, cpp), :], sidx_all)
      pltpu.sync_copy(dst_hbm.at[pl.ds(t * _CPT + ph * cpp, cpp), :], didx_all)

      pltpu.async_copy(feat_hbm.at[sidx_all.at[0]], rbuf0, gsem0)
      pltpu.async_copy(feat_hbm.at[sidx_all.at[1]], rbuf1, gsem1)

      def pair(p, carry):
        for b in range(2):
          j = 2 * p + b
          rb, gs, ss, ds = rbufs[b], gsems[b], ssems[b], dsems[b]
          pltpu.make_async_copy(feat_hbm.at[pl.ds(0, _CHUNK), :], rb, gs).wait()
          pltpu.async_copy(rb, acc.at[didx_all.at[j]], ss, add=True)
          pltpu.async_copy(ones, deg.at[didx_all.at[j]], ds, add=True)
          pltpu.make_async_copy(rb, acc.at[pl.ds(0, _CHUNK), :], ss).wait()
          pltpu.make_async_copy(ones, deg.at[pl.ds(0, _CHUNK)], ds).wait()
          @pl.when(j + 2 < cpp)
          def _():
            pltpu.async_copy(feat_hbm.at[sidx_all.at[j + 2]], rb, gs)
        return carry

      lax.fori_loop(0, cpp // 2, pair, 0)
    plsc.subcore_barrier()

    pltpu.sync_copy(acc.at[pl.ds(base, rows_per_tile), :],
                    psum_out.at[c, pl.ds(base, rows_per_tile), :])
    pltpu.sync_copy(deg.at[pl.ds(s * rows_per_tile, rows_per_tile)],
                    pdeg_out.at[c, pl.ds(s * rows_per_tile, rows_per_tile)])

  return body(feat_src, src2d, dst2d, jnp.zeros((_NP, _D), _f32))


def _combine(psum, pdeg, h_self):
  p0, p1 = psum[0], psum[1]
  d0 = pdeg[0].reshape(-1, 1)
  d1 = pdeg[1].reshape(-1, 1)
  rows = 1000
  grid = (_N // rows,)

  def body(p0_ref, p1_ref, d0_ref, d1_ref, h_ref, o_ref):
    degree = jnp.maximum(d0_ref[...] + d1_ref[...], 1.0)
    o_ref[...] = jnp.maximum(
        (p0_ref[...] + p1_ref[...]) / degree + h_ref[...], 0.0)

  return pl.pallas_call(
      body,
      grid=grid,
      in_specs=[
          pl.BlockSpec((rows, _D), lambda i: (i, 0)),
          pl.BlockSpec((rows, _D), lambda i: (i, 0)),
          pl.BlockSpec((rows, 1), lambda i: (i, 0)),
          pl.BlockSpec((rows, 1), lambda i: (i, 0)),
          pl.BlockSpec((rows, _D), lambda i: (i, 0)),
      ],
      out_specs=pl.BlockSpec((rows, _D), lambda i: (i, 0)),
      out_shape=jax.ShapeDtypeStruct((_N, _D), _f32),
  )(p0, p1, d0, d1, h_self)


def kernel(feat_src, h_self, edge_index):
  npad = _EPAD - _E
  src2d = jnp.concatenate(
      [edge_index[0], jnp.zeros((npad,), jnp.int32)]).reshape(-1, _CHUNK)
  dst2d = jnp.concatenate(
      [edge_index[1], jnp.full((npad,), _NP - 1, jnp.int32)]).reshape(-1, _CHUNK)
  psum, pdeg = _sc_partials(feat_src, src2d, dst2d)
  return _combine(psum, pdeg, h_self)


# speedup vs baseline: 1.0076x; 1.0076x over previous
import functools

import jax
import jax.numpy as jnp
from jax import lax
from jax.experimental import pallas as pl
from jax.experimental.pallas import tpu as pltpu
from jax.experimental.pallas import tpu_sc as plsc

_N = 10000
_E = 320000
_D = 128
_CHUNK = 128
_NCHUNKS = _E // _CHUNK
_NP = 10240
_NC = 2
_NS = 16

_f32 = jnp.float32


_EPAD = 2560 * _CHUNK
_CPT = (_EPAD // _CHUNK) // (_NC * _NS)


def _sc_partials(feat_src, src2d, dst2d):
  mesh = plsc.VectorSubcoreMesh(core_axis_name="c", subcore_axis_name="s")
  rows_per_tile = _NP // _NS

  @functools.partial(
      pl.kernel,
      mesh=mesh,
      out_type=(
          jax.ShapeDtypeStruct((_NC, _NP, _D), _f32),
          jax.ShapeDtypeStruct((_NC, _NP), _f32),
      ),
      scratch_types=[
          pltpu.VMEM((_CHUNK, _D), _f32),
          pltpu.VMEM((_CHUNK, _D), _f32),
          pltpu.VMEM((_CHUNK,), jnp.int32),
          pltpu.VMEM((_CHUNK,), jnp.int32),
          pltpu.VMEM((1, _CHUNK), jnp.int32),
          pltpu.VMEM((1, _CHUNK), jnp.int32),
          pltpu.VMEM((_CHUNK,), _f32),
          pltpu.VMEM((rows_per_tile,), _f32),
          pltpu.VMEM_SHARED((_NP, _D), _f32),
          pltpu.VMEM_SHARED((_NP,), _f32),
          pltpu.SemaphoreType.DMA,
          pltpu.SemaphoreType.DMA,
      ],
  )
  def body(feat_hbm, src_hbm, dst_hbm, zeros_hbm, psum_out, pdeg_out,
           rbuf0, rbuf1, sidx0, sidx1, didx0, didx1, ones, zdeg, acc, deg,
           gsem0, gsem1):
    c = lax.axis_index("c")
    s = lax.axis_index("s")
    t = c * _NS + s

    for i in range(_CHUNK // 16):
      ones[pl.ds(16 * i, 16)] = jnp.full((16,), 1.0, _f32)
    for i in range(rows_per_tile // 16):
      zdeg[pl.ds(16 * i, 16)] = jnp.zeros((16,), _f32)

    base = s * rows_per_tile
    pltpu.sync_copy(zeros_hbm.at[pl.ds(base, rows_per_tile), :],
                    acc.at[pl.ds(base, rows_per_tile), :])
    pltpu.sync_copy(zdeg, deg.at[pl.ds(s * rows_per_tile, rows_per_tile)])
    plsc.subcore_barrier()

    rbufs = (rbuf0, rbuf1)
    sidxs = (sidx0, sidx1)
    didxs = (didx0, didx1)
    gsems = (gsem0, gsem1)
    c0 = t * _CPT

    for b in range(2):
      pltpu.sync_copy(src_hbm.at[c0 + b], sidxs[b])
      pltpu.sync_copy(dst_hbm.at[c0 + b], didxs[b].at[0])
      pltpu.async_copy(feat_hbm.at[sidxs[b]], rbufs[b], gsems[b])

    def pair(p, carry):
      for b in range(2):
        j = 2 * p + b
        rb, si, di, gs = rbufs[b], sidxs[b], didxs[b], gsems[b]
        pltpu.make_async_copy(feat_hbm.at[pl.ds(0, _CHUNK), :], rb, gs).wait()
        pltpu.sync_copy(rb, acc.at[di.at[0]], add=True)
        pltpu.sync_copy(ones, deg.at[di.at[0]], add=True)
        @pl.when(j + 2 < _CPT)
        def _():
          pltpu.sync_copy(src_hbm.at[c0 + j + 2], si)
          pltpu.sync_copy(dst_hbm.at[c0 + j + 2], di.at[0])
          pltpu.async_copy(feat_hbm.at[si], rb, gs)
      return carry

    lax.fori_loop(0, _CPT // 2, pair, 0)
    plsc.subcore_barrier()

    pltpu.sync_copy(acc.at[pl.ds(base, rows_per_tile), :],
                    psum_out.at[c, pl.ds(base, rows_per_tile), :])
    pltpu.sync_copy(deg.at[pl.ds(s * rows_per_tile, rows_per_tile)],
                    pdeg_out.at[c, pl.ds(s * rows_per_tile, rows_per_tile)])

  return body(feat_src, src2d, dst2d, jnp.zeros((_NP, _D), _f32))


def _combine(psum, pdeg, h_self):
  p0, p1 = psum[0], psum[1]
  d0 = pdeg[0].reshape(-1, 1)
  d1 = pdeg[1].reshape(-1, 1)
  rows = 1000
  grid = (_N // rows,)

  def body(p0_ref, p1_ref, d0_ref, d1_ref, h_ref, o_ref):
    degree = jnp.maximum(d0_ref[...] + d1_ref[...], 1.0)
    o_ref[...] = jnp.maximum(
        (p0_ref[...] + p1_ref[...]) / degree + h_ref[...], 0.0)

  return pl.pallas_call(
      body,
      grid=grid,
      in_specs=[
          pl.BlockSpec((rows, _D), lambda i: (i, 0)),
          pl.BlockSpec((rows, _D), lambda i: (i, 0)),
          pl.BlockSpec((rows, 1), lambda i: (i, 0)),
          pl.BlockSpec((rows, 1), lambda i: (i, 0)),
          pl.BlockSpec((rows, _D), lambda i: (i, 0)),
      ],
      out_specs=pl.BlockSpec((rows, _D), lambda i: (i, 0)),
      out_shape=jax.ShapeDtypeStruct((_N, _D), _f32),
  )(p0, p1, d0, d1, h_self)


def kernel(feat_src, h_self, edge_index):
  npad = _EPAD - _E
  src2d = jnp.concatenate(
      [edge_index[0], jnp.zeros((npad,), jnp.int32)]).reshape(-1, _CHUNK)
  dst2d = jnp.concatenate(
      [edge_index[1], jnp.full((npad,), _NP - 1, jnp.int32)]).reshape(-1, _CHUNK)
  psum, pdeg = _sc_partials(feat_src, src2d, dst2d)
  return _combine(psum, pdeg, h_self)
